# Initial kernel scaffold; baseline (speedup 1.0000x reference)
import jax
import jax.numpy as jnp
from jax.experimental import pallas as pl


def kernel(params, input_ids, decoder_input_ids):
    raise NotImplementedError("write your pallas kernel here")


# speedup vs baseline: 1.5139x; 1.5139x over previous
import functools

import jax
import jax.numpy as jnp
from jax import lax
from jax.experimental import pallas as pl
from jax.experimental.pallas import tpu as pltpu
from jax.experimental.pallas import tpu_sc as plsc

D = 768
VOCAB = 32128
K_HEAVY = 32
N_HEADS = 12
LIGHT_DIM = 64
T = 2048
DH = D // N_HEADS

F32 = jnp.float32


def _fiota(shape, dim):
    return lax.broadcasted_iota(jnp.int32, shape, dim).astype(F32)


def _ln_v(x):
    mu = jnp.mean(x, axis=-1, keepdims=True)
    var = jnp.mean((x - mu) * (x - mu), axis=-1, keepdims=True)
    return (x - mu) / jnp.sqrt(var + 1e-5)


def _topk_oh(s_row, kk):
    n = s_row.shape[1]
    iota_t = _fiota((1, n), 1)
    iota_kt = _fiota((kk, n), 1)
    rows_k = _fiota((kk, 1), 0)
    cols_k = _fiota((1, kk), 1)

    def body(i, c):
        s, oh, vals_c, idxs_c, idxs_r = c
        fi = i.astype(F32)
        m = jnp.max(s)
        idx = jnp.min(jnp.where(s == m, iota_t, F32(n)))
        s = jnp.where(iota_t == idx, F32(-1e30), s)
        rowsel = (rows_k == fi).astype(F32)
        colsel = (cols_k == fi).astype(F32)
        oh = oh + rowsel * (iota_kt == idx).astype(F32)
        vals_c = vals_c + rowsel * m
        idxs_c = idxs_c + rowsel * idx
        idxs_r = idxs_r + colsel * idx
        return s, oh, vals_c, idxs_c, idxs_r

    init = (s_row,
            jnp.zeros((kk, n), F32),
            jnp.zeros((kk, 1), F32),
            jnp.zeros((kk, 1), F32),
            jnp.zeros((1, kk), F32))
    _, oh, vals_c, idxs_c, idxs_r = lax.fori_loop(0, kk, body, init)
    return oh, vals_c, idxs_c, idxs_r


def _dotT(a, b):
    return lax.dot_general(a, b, (((1,), (1,)), ((), ())),
                           preferred_element_type=F32)


def _dotL(a, b):
    return lax.dot_general(a, b, (((0,), (0,)), ((), ())),
                           preferred_element_type=F32)


def _mm(a, b):
    return jnp.dot(a, b, preferred_element_type=F32)


def _row_scores(rv, h):
    return lax.dot_general(rv, h, (((0,), (1,)), ((), ())),
                           preferred_element_type=F32)


def _heavy_delta(h_q, h_kv, rq, rkv, wq, wk, wv, wo, causal):
    sq = _row_scores(rq, h_q)
    skv = _row_scores(rkv, h_kv)
    oh_q, qv_c, qi_c, _ = _topk_oh(sq, K_HEAVY)
    oh_kv, kvv_c, _, kvi_r = _topk_oh(skv, K_HEAVY)
    qs = _mm(oh_q, h_q)
    ks = _mm(oh_kv, h_kv) * jax.nn.sigmoid(kvv_c)
    q = _mm(qs, wq)
    k = _mm(ks, wk)
    v = _mm(ks, wv)
    outs = []
    for hh in range(N_HEADS):
        sl = slice(hh * DH, (hh + 1) * DH)
        lg = _dotT(q[:, sl], k[:, sl]) * (1.0 / 8.0)
        if causal:
            lg = jnp.where(kvi_r <= qi_c, lg, F32(-1e9))
        a = jax.nn.softmax(lg, axis=-1)
        outs.append(_mm(a, v[:, sl]))
    out = jnp.concatenate(outs, axis=1)
    out = _mm(out, wo) * jax.nn.sigmoid(qv_c)
    return _dotL(oh_q, out)


def _self_body(x_ref, lq, lk, lv, lo, rq, rkv, hq, hk, hv, ho, o_ref, *,
               causal):
    x = x_ref[...]
    h = _ln_v(x)
    q = _mm(h, lq[...])
    k = _mm(h, lk[...])
    v = _mm(h, lv[...])
    lg = _dotT(q, k) * (1.0 / 8.0)
    if causal:
        ri = _fiota((x.shape[0], 1), 0)
        ci = _fiota((1, x.shape[0]), 1)
        lg = jnp.where(ci <= ri, lg, F32(-1e9))
    a = jax.nn.softmax(lg, axis=-1)
    x = x + _mm(_mm(a, v), lo[...])
    x = x + _heavy_delta(h, h, rq[...], rkv[...], hq[...], hk[...], hv[...],
                         ho[...], causal)
    o_ref[...] = x


def _cross_body(x_ref, enc_ref, clq, clk, clv, clo, crq, crkv, cq, ck, cv,
                co, o_ref):
    x = x_ref[...]
    h = _ln_v(x)
    e = _ln_v(enc_ref[...])
    q = _mm(h, clq[...])
    k = _mm(e, clk[...])
    v = _mm(e, clv[...])
    lg = _dotT(q, k) * (1.0 / 8.0)
    a = jax.nn.softmax(lg, axis=-1)
    x = x + _mm(_mm(a, v), clo[...])
    x = x + _heavy_delta(h, e, crq[...], crkv[...], cq[...], ck[...], cv[...],
                         co[...], False)
    o_ref[...] = x


def _ff_body(x_ref, lf1, lf2, rf, hf1, hf2, o_ref):
    x = x_ref[...]
    h2 = _ln_v(x)
    x = x + _mm(jax.nn.gelu(_mm(h2, lf1[...])), lf2[...])
    s = _row_scores(rf[...], h2)
    oh, fv_c, _, _ = _topk_oh(s, K_HEAVY)
    sel = _mm(oh, h2)
    hff = _mm(jax.nn.gelu(_mm(sel, hf1[...])), hf2[...]) * jax.nn.sigmoid(fv_c)
    o_ref[...] = x + _dotL(oh, hff)


def _lm_body(x_ref, w_ref, b_ref, o_ref):
    o_ref[...] = _mm(x_ref[...], w_ref[...]) + b_ref[...]


def _col(v):
    return v.reshape(v.shape[0], 1)


def _self_block(x, p, causal):
    return pl.pallas_call(
        functools.partial(_self_body, causal=causal),
        out_shape=jax.ShapeDtypeStruct((T, D), F32),
    )(x, p['lq'], p['lk'], p['lv'], p['lo'], _col(p['rq']), _col(p['rkv']),
      p['hq'], p['hk'], p['hv'], p['ho'])


def _cross_block(x, enc, p):
    return pl.pallas_call(
        _cross_body,
        out_shape=jax.ShapeDtypeStruct((T, D), F32),
    )(x, enc, p['clq'], p['clk'], p['clv'], p['clo'], _col(p['crq']),
      _col(p['crkv']), p['cq'], p['ck'], p['cv'], p['co'])


def _ff_block(x, p):
    return pl.pallas_call(
        _ff_body,
        out_shape=jax.ShapeDtypeStruct((T, D), F32),
    )(x, p['lf1'], p['lf2'], _col(p['rf']), p['hf1'], p['hf2'])


_LM_TILE = 512


def _lm_head(y, w, b):
    nt = pl.cdiv(VOCAB, _LM_TILE)
    return pl.pallas_call(
        _lm_body,
        grid=(nt,),
        in_specs=[
            pl.BlockSpec((T, D), lambda i: (0, 0)),
            pl.BlockSpec((D, _LM_TILE), lambda i: (0, i)),
            pl.BlockSpec((1, _LM_TILE), lambda i: (0, i)),
        ],
        out_specs=pl.BlockSpec((T, _LM_TILE), lambda i: (0, i)),
        out_shape=jax.ShapeDtypeStruct((T, VOCAB), F32),
    )(y, w, b.reshape(1, VOCAB))


_SC_NC = 2
_SC_NS = 16
_SC_NW = _SC_NC * _SC_NS
_B_PER_W = T // _SC_NW

_CACHE = {}


def _build_sc_gather():
    mesh = plsc.VectorSubcoreMesh(core_axis_name="c", subcore_axis_name="s")

    @functools.partial(
        pl.kernel, mesh=mesh,
        out_type=jax.ShapeDtypeStruct((T, D), F32),
        scratch_types=[
            pltpu.VMEM((_B_PER_W,), jnp.int32),
            pltpu.VMEM((_B_PER_W, D), F32),
            pltpu.SemaphoreType.DMA,
        ],
    )
    def gather_k(table_hbm, idx_hbm, out_hbm, idx_v, rows_v, sem):
        wid = lax.axis_index("s") * _SC_NC + lax.axis_index("c")
        base = wid * _B_PER_W
        pltpu.sync_copy(idx_hbm.at[pl.ds(base, _B_PER_W)], idx_v)
        pltpu.async_copy(table_hbm.at[idx_v], rows_v, sem).wait()
        pltpu.sync_copy(rows_v, out_hbm.at[pl.ds(base, _B_PER_W)])

    return gather_k


def _embed_lookup(table, ids):
    f = _CACHE.get('gather')
    if f is None:
        f = _build_sc_gather()
        _CACHE['gather'] = f
    return f(table, ids)


def kernel(params, input_ids, decoder_input_ids):
    p = params
    x = _embed_lookup(p['enc_embed'], input_ids.reshape(T))
    for lp in p['enc_layers']:
        x = _self_block(x, lp, causal=False)
        x = _ff_block(x, lp)
    enc = x
    y = _embed_lookup(p['dec_embed'], decoder_input_ids.reshape(T))
    for lp in p['dec_layers']:
        y = _self_block(y, lp, causal=True)
        y = _cross_block(y, enc, lp)
        y = _ff_block(y, lp)
    out = _lm_head(y, p['lm_w'], p['lm_b'])
    return out.reshape(1, T, VOCAB)


# speedup vs baseline: 1.5895x; 1.0499x over previous
import functools

import jax
import jax.numpy as jnp
from jax import lax
from jax.experimental import pallas as pl
from jax.experimental.pallas import tpu as pltpu
from jax.experimental.pallas import tpu_sc as plsc

D = 768
VOCAB = 32128
K_HEAVY = 32
N_HEADS = 12
LIGHT_DIM = 64
T = 2048
DH = D // N_HEADS

F32 = jnp.float32


def _fiota(shape, dim):
    return lax.broadcasted_iota(jnp.int32, shape, dim).astype(F32)


def _ln_v(x):
    mu = jnp.mean(x, axis=-1, keepdims=True)
    var = jnp.mean((x - mu) * (x - mu), axis=-1, keepdims=True)
    return (x - mu) / jnp.sqrt(var + 1e-5)


def _topk_oh(s_row, kk):
    n = s_row.shape[1]
    iota_t = _fiota((1, n), 1)
    iota_kt = _fiota((kk, n), 1)
    rows_k = _fiota((kk, 1), 0)
    cols_k = _fiota((1, kk), 1)

    def body(i, c):
        s, vals_c, idxs_c, idxs_r = c
        fi = i.astype(F32)
        m = jnp.max(s)
        idx = jnp.min(jnp.where(s == m, iota_t, F32(n)))
        s = jnp.where(iota_t == idx, F32(-1e30), s)
        rowsel = (rows_k == fi).astype(F32)
        colsel = (cols_k == fi).astype(F32)
        vals_c = vals_c + rowsel * m
        idxs_c = idxs_c + rowsel * idx
        idxs_r = idxs_r + colsel * idx
        return s, vals_c, idxs_c, idxs_r

    init = (s_row,
            jnp.zeros((kk, 1), F32),
            jnp.zeros((kk, 1), F32),
            jnp.zeros((1, kk), F32))
    _, vals_c, idxs_c, idxs_r = lax.fori_loop(0, kk, body, init)
    oh = (iota_kt == idxs_c).astype(F32)
    return oh, vals_c, idxs_c, idxs_r


def _dotT(a, b):
    return lax.dot_general(a, b, (((1,), (1,)), ((), ())),
                           preferred_element_type=F32)


def _dotL(a, b):
    return lax.dot_general(a, b, (((0,), (0,)), ((), ())),
                           preferred_element_type=F32)


def _mm(a, b):
    return jnp.dot(a, b, preferred_element_type=F32)


def _row_scores(rv, h):
    return lax.dot_general(rv, h, (((0,), (1,)), ((), ())),
                           preferred_element_type=F32)


def _heavy_delta(h_q, h_kv, rq, rkv, wq, wk, wv, wo, causal):
    sq = _row_scores(rq, h_q)
    skv = _row_scores(rkv, h_kv)
    oh_q, qv_c, qi_c, _ = _topk_oh(sq, K_HEAVY)
    oh_kv, kvv_c, _, kvi_r = _topk_oh(skv, K_HEAVY)
    qs = _mm(oh_q, h_q)
    ks = _mm(oh_kv, h_kv) * jax.nn.sigmoid(kvv_c)
    q = _mm(qs, wq)
    k = _mm(ks, wk)
    v = _mm(ks, wv)
    outs = []
    for hh in range(N_HEADS):
        sl = slice(hh * DH, (hh + 1) * DH)
        lg = _dotT(q[:, sl], k[:, sl]) * (1.0 / 8.0)
        if causal:
            lg = jnp.where(kvi_r <= qi_c, lg, F32(-1e9))
        a = jax.nn.softmax(lg, axis=-1)
        outs.append(_mm(a, v[:, sl]))
    out = jnp.concatenate(outs, axis=1)
    out = _mm(out, wo) * jax.nn.sigmoid(qv_c)
    return _dotL(oh_q, out)


def _self_body(x_ref, lq, lk, lv, lo, rq, rkv, hq, hk, hv, ho, o_ref, *,
               causal):
    x = x_ref[...]
    h = _ln_v(x)
    q = _mm(h, lq[...])
    k = _mm(h, lk[...])
    v = _mm(h, lv[...])
    lg = _dotT(q, k) * (1.0 / 8.0)
    if causal:
        ri = _fiota((x.shape[0], 1), 0)
        ci = _fiota((1, x.shape[0]), 1)
        lg = jnp.where(ci <= ri, lg, F32(-1e9))
    a = jax.nn.softmax(lg, axis=-1)
    x = x + _mm(_mm(a, v), lo[...])
    x = x + _heavy_delta(h, h, rq[...], rkv[...], hq[...], hk[...], hv[...],
                         ho[...], causal)
    o_ref[...] = x


def _cross_body(x_ref, enc_ref, clq, clk, clv, clo, crq, crkv, cq, ck, cv,
                co, o_ref):
    x = x_ref[...]
    h = _ln_v(x)
    e = _ln_v(enc_ref[...])
    q = _mm(h, clq[...])
    k = _mm(e, clk[...])
    v = _mm(e, clv[...])
    lg = _dotT(q, k) * (1.0 / 8.0)
    a = jax.nn.softmax(lg, axis=-1)
    x = x + _mm(_mm(a, v), clo[...])
    x = x + _heavy_delta(h, e, crq[...], crkv[...], cq[...], ck[...], cv[...],
                         co[...], False)
    o_ref[...] = x


def _ff_body(x_ref, lf1, lf2, rf, hf1, hf2, o_ref):
    x = x_ref[...]
    h2 = _ln_v(x)
    x = x + _mm(jax.nn.gelu(_mm(h2, lf1[...])), lf2[...])
    s = _row_scores(rf[...], h2)
    oh, fv_c, _, _ = _topk_oh(s, K_HEAVY)
    sel = _mm(oh, h2)
    hff = _mm(jax.nn.gelu(_mm(sel, hf1[...])), hf2[...]) * jax.nn.sigmoid(fv_c)
    o_ref[...] = x + _dotL(oh, hff)


def _lm_body(x_ref, w_ref, b_ref, o_ref):
    o_ref[...] = _mm(x_ref[...], w_ref[...]) + b_ref[...]


def _col(v):
    return v.reshape(v.shape[0], 1)


def _self_block(x, p, causal):
    return pl.pallas_call(
        functools.partial(_self_body, causal=causal),
        out_shape=jax.ShapeDtypeStruct((T, D), F32),
    )(x, p['lq'], p['lk'], p['lv'], p['lo'], _col(p['rq']), _col(p['rkv']),
      p['hq'], p['hk'], p['hv'], p['ho'])


def _cross_block(x, enc, p):
    return pl.pallas_call(
        _cross_body,
        out_shape=jax.ShapeDtypeStruct((T, D), F32),
    )(x, enc, p['clq'], p['clk'], p['clv'], p['clo'], _col(p['crq']),
      _col(p['crkv']), p['cq'], p['ck'], p['cv'], p['co'])


def _ff_block(x, p):
    return pl.pallas_call(
        _ff_body,
        out_shape=jax.ShapeDtypeStruct((T, D), F32),
    )(x, p['lf1'], p['lf2'], _col(p['rf']), p['hf1'], p['hf2'])


_LM_TILE = 512


def _lm_head(y, w, b):
    nt = pl.cdiv(VOCAB, _LM_TILE)
    return pl.pallas_call(
        _lm_body,
        grid=(nt,),
        in_specs=[
            pl.BlockSpec((T, D), lambda i: (0, 0)),
            pl.BlockSpec((D, _LM_TILE), lambda i: (0, i)),
            pl.BlockSpec((1, _LM_TILE), lambda i: (0, i)),
        ],
        out_specs=pl.BlockSpec((T, _LM_TILE), lambda i: (0, i)),
        out_shape=jax.ShapeDtypeStruct((T, VOCAB), F32),
    )(y, w, b.reshape(1, VOCAB))


_SC_NC = 2
_SC_NS = 16
_SC_NW = _SC_NC * _SC_NS
_B_PER_W = T // _SC_NW

_CACHE = {}


def _build_sc_gather():
    mesh = plsc.VectorSubcoreMesh(core_axis_name="c", subcore_axis_name="s")

    @functools.partial(
        pl.kernel, mesh=mesh,
        out_type=jax.ShapeDtypeStruct((T, D), F32),
        scratch_types=[
            pltpu.VMEM((_B_PER_W,), jnp.int32),
            pltpu.VMEM((_B_PER_W, D), F32),
            pltpu.SemaphoreType.DMA,
        ],
    )
    def gather_k(table_hbm, idx_hbm, out_hbm, idx_v, rows_v, sem):
        wid = lax.axis_index("s") * _SC_NC + lax.axis_index("c")
        base = wid * _B_PER_W
        pltpu.sync_copy(idx_hbm.at[pl.ds(base, _B_PER_W)], idx_v)
        pltpu.async_copy(table_hbm.at[idx_v], rows_v, sem).wait()
        pltpu.sync_copy(rows_v, out_hbm.at[pl.ds(base, _B_PER_W)])

    return gather_k


def _embed_lookup(table, ids):
    f = _CACHE.get('gather')
    if f is None:
        f = _build_sc_gather()
        _CACHE['gather'] = f
    return f(table, ids)


def kernel(params, input_ids, decoder_input_ids):
    p = params
    x = _embed_lookup(p['enc_embed'], input_ids.reshape(T))
    for lp in p['enc_layers']:
        x = _self_block(x, lp, causal=False)
        x = _ff_block(x, lp)
    enc = x
    y = _embed_lookup(p['dec_embed'], decoder_input_ids.reshape(T))
    for lp in p['dec_layers']:
        y = _self_block(y, lp, causal=True)
        y = _cross_block(y, enc, lp)
        y = _ff_block(y, lp)
    out = _lm_head(y, p['lm_w'], p['lm_b'])
    return out.reshape(1, T, VOCAB)
